# baseline (speedup 1.0000x reference)
import functools

import jax
import jax.numpy as jnp
from jax import lax
from jax.experimental import pallas as pl
from jax.experimental.pallas import tpu as pltpu
from jax.experimental.pallas import tpu_sc as plsc

N = 10000
E = 320000
F_IN = 128
DIM = 64
C = 16

NC = 2
NS = 16
NW = NC * NS
K = 128
SLOTS = 80
NCHUNK_PAD = SLOTS * NW
E_PAD = NCHUNK_PAD * K
NPAD = 10112
ZCH = NPAD // K
ZSLOTS = -(-ZCH // NS)


def _make_agg(F, NBUF):
    mesh = plsc.VectorSubcoreMesh(core_axis_name="c", subcore_axis_name="s")

    @functools.partial(
        pl.kernel,
        out_type=jax.ShapeDtypeStruct((NC, NPAD, F), jnp.float32),
        mesh=mesh,
        scratch_types=[
            pltpu.VMEM((NBUF, K), jnp.int32),
            pltpu.VMEM((NBUF, K), jnp.int32),
            pltpu.VMEM((NBUF, K, F), jnp.float32),
            pltpu.VMEM_SHARED((NPAD, F), jnp.float32),
            pltpu.SemaphoreType.DMA,
            pltpu.SemaphoreType.DMA,
        ],
        compiler_params=pltpu.CompilerParams(use_tc_tiling_on_sc=False),
    )
    def agg_kernel(x_hbm, edge_hbm, out_hbm, src_v, dst_v, rows_v,
                   acc_sh, sem_i, sem_g):
        cid = lax.axis_index("c")
        sid = lax.axis_index("s")
        wid = sid * NC + cid

        zvec = jnp.zeros((16,), jnp.float32)

        def zero_row(i, _):
            for j in range(F // 16):
                rows_v[0, i, pl.ds(16 * j, 16)] = zvec
            return 0

        lax.fori_loop(0, K, zero_row, 0)

        def zero_acc(c, _):
            cc = sid + NS * c

            @pl.when(cc < ZCH)
            def _():
                pltpu.sync_copy(rows_v.at[0], acc_sh.at[pl.ds(cc * K, K)])

            return 0

        lax.fori_loop(0, ZSLOTS, zero_acc, 0)
        plsc.subcore_barrier()

        NGROUPS = SLOTS // NBUF

        def do_group(g, _):
            c0 = wid * SLOTS + g * NBUF
            di_s = pltpu.async_copy(edge_hbm.at[0, pl.ds(c0, NBUF)], src_v,
                                    sem_i)
            di_d = pltpu.async_copy(edge_hbm.at[1, pl.ds(c0, NBUF)], dst_v,
                                    sem_i)
            di_s.wait()
            di_d.wait()
            gathers = []
            for b in range(NBUF):
                gathers.append(
                    pltpu.async_copy(x_hbm.at[src_v.at[b]], rows_v.at[b],
                                     sem_g))
            for b in range(NBUF):
                gathers[b].wait()
                pltpu.sync_copy(rows_v.at[b], acc_sh.at[dst_v.at[b]],
                                add=True)
            return 0

        lax.fori_loop(0, NGROUPS, do_group, 0)
        plsc.subcore_barrier()

        def copy_out(c, _):
            cc = sid + NS * c

            @pl.when(cc < ZCH)
            def _():
                pltpu.sync_copy(acc_sh.at[pl.ds(cc * K, K)],
                                out_hbm.at[cid, pl.ds(cc * K, K)])

            return 0

        lax.fori_loop(0, ZSLOTS, copy_out, 0)

    return agg_kernel


def _make_mm(F_in, F_out):
    def mm_body(x_ref, a_ref, w_ref, b_ref, o_ref):
        h = x_ref[...] + a_ref[0, :N, :] + a_ref[1, :N, :]
        o_ref[...] = (
            jnp.dot(h, w_ref[...], preferred_element_type=jnp.float32)
            + b_ref[...]
        )

    return pl.pallas_call(
        mm_body,
        out_shape=jax.ShapeDtypeStruct((N, F_out), jnp.float32),
        in_specs=[
            pl.BlockSpec(memory_space=pltpu.VMEM),
            pl.BlockSpec(memory_space=pltpu.VMEM),
            pl.BlockSpec(memory_space=pltpu.VMEM),
            pl.BlockSpec(memory_space=pltpu.VMEM),
        ],
        out_specs=pl.BlockSpec(memory_space=pltpu.VMEM),
    )


_agg128 = _make_agg(F_IN, 2)
_agg64 = _make_agg(DIM, 8)
_mm1 = _make_mm(F_IN, DIM)
_mm_mid = _make_mm(DIM, DIM)
_mm5 = _make_mm(DIM, C)


def kernel(x, edge_index, W1, b1, W2, b2, W3, b3, W4, b4, W5, b5):
    edge_index = edge_index.astype(jnp.int32)
    npad_e = E_PAD - E
    pad_src = jnp.zeros((npad_e,), jnp.int32)
    pad_dst = N + (jnp.arange(npad_e, dtype=jnp.int32) % (NPAD - N))
    edges = jnp.concatenate(
        [edge_index, jnp.stack([pad_src, pad_dst])], axis=1
    ).reshape(2, NCHUNK_PAD, K)

    def layer(agg_fn, mm_fn, h, W, b):
        parts = agg_fn(h, edges)
        return mm_fn(h, parts, W, b.reshape(1, -1))

    h = layer(_agg128, _mm1, x, W1, b1)
    h = layer(_agg64, _mm_mid, h, W2, b2)
    h = layer(_agg64, _mm_mid, h, W3, b3)
    h = layer(_agg64, _mm_mid, h, W4, b4)
    h = layer(_agg64, _mm5, h, W5, b5)
    return h
